# Initial kernel scaffold; baseline (speedup 1.0000x reference)
import jax
import jax.numpy as jnp
from jax.experimental import pallas as pl


def kernel(input_ids, table):
    raise NotImplementedError("write your pallas kernel here")


# speedup vs baseline: 5.1391x; 5.1391x over previous
import functools

import jax
import jax.numpy as jnp
from jax import lax
from jax.experimental import pallas as pl
from jax.experimental.pallas import tpu as pltpu
from jax.experimental.pallas import tpu_sc as plsc

DIM = 4
NUM_WORKERS = 32
CHUNK = 10240


@functools.lru_cache(maxsize=None)
def _emb_fn(n_total, vocab):
    n_per_w = n_total // NUM_WORKERS
    chunk = CHUNK if n_per_w % CHUNK == 0 else n_per_w
    n_chunks = n_per_w // chunk
    mesh = plsc.VectorSubcoreMesh(core_axis_name="c", subcore_axis_name="s")

    @functools.partial(
        pl.kernel,
        mesh=mesh,
        out_type=jax.ShapeDtypeStruct((n_total * DIM,), jnp.float32),
        scratch_types=[
            pltpu.VMEM((vocab * DIM,), jnp.float32),
            pltpu.VMEM((chunk,), jnp.int32),
            pltpu.VMEM((chunk * DIM,), jnp.float32),
        ],
        compiler_params=pltpu.CompilerParams(needs_layout_passes=False),
    )
    def emb(ids_hbm, table_hbm, out_hbm, table_v, ids_v, out_v):
        wid = lax.axis_index("s") * 2 + lax.axis_index("c")
        pltpu.sync_copy(table_hbm, table_v)
        lane4 = lax.iota(jnp.int32, 16) * DIM
        base_w = wid * n_per_w

        def chunk_body(c, carry):
            base = base_w + c * chunk
            pltpu.sync_copy(ids_hbm.at[pl.ds(base, chunk)], ids_v)

            def inner(i, carry2):
                idv = ids_v[pl.ds(i * 16, 16)]
                t4 = idv * DIM
                ob = i * (16 * DIM)
                for d in range(DIM):
                    vals = plsc.load_gather(table_v, [t4 + d])
                    plsc.store_scatter(out_v, [ob + lane4 + d], vals)
                return carry2

            lax.fori_loop(0, chunk // 16, inner, 0)
            pltpu.sync_copy(out_v, out_hbm.at[pl.ds(base * DIM, chunk * DIM)])
            return carry

        lax.fori_loop(0, n_chunks, chunk_body, 0)

    return emb


def kernel(input_ids, table):
    n_total = 1
    for s in input_ids.shape:
        n_total *= s
    ids_flat = input_ids.reshape(-1).astype(jnp.int32)
    table_flat = table.reshape(-1)
    out = _emb_fn(n_total, table.shape[0])(ids_flat, table_flat)
    return out.reshape(input_ids.shape + (DIM,))


# speedup vs baseline: 54.6194x; 10.6281x over previous
import functools

import jax
import jax.numpy as jnp
from jax import lax
from jax.experimental import pallas as pl
from jax.experimental.pallas import tpu as pltpu
from jax.experimental.pallas import tpu_sc as plsc

DIM = 4
NUM_WORKERS = 32
LANES = 128
CHUNK = 10240


@functools.lru_cache(maxsize=None)
def _emb_fn(n_total, vocab):
    n_per_w = n_total // NUM_WORKERS
    chunk = CHUNK if n_per_w % CHUNK == 0 else n_per_w
    n_chunks = n_per_w // chunk
    groups = chunk // 16
    mesh = plsc.VectorSubcoreMesh(core_axis_name="c", subcore_axis_name="s")

    @functools.partial(
        pl.kernel,
        mesh=mesh,
        out_type=jax.ShapeDtypeStruct((n_total * DIM,), jnp.float32),
        scratch_types=[
            pltpu.VMEM((vocab * DIM,), jnp.float32),
            pltpu.VMEM((chunk,), jnp.int32),
            pltpu.VMEM((chunk * DIM,), jnp.float32),
        ],
        compiler_params=pltpu.CompilerParams(needs_layout_passes=False),
    )
    def emb(ids_hbm, table_hbm, out_hbm, table_v, ids_v, out_v):
        wid = lax.axis_index("s") * 2 + lax.axis_index("c")
        pltpu.sync_copy(table_hbm, table_v)
        base_w = wid * n_per_w

        def chunk_body(c, carry):
            base = base_w + c * chunk
            pltpu.sync_copy(ids_hbm.at[pl.ds(base, chunk)], ids_v)

            def g_body(g, carry2):
                idv = ids_v[pl.ds(g * 16, 16)]
                t4 = idv * DIM
                ob = (g >> 3) * (DIM * LANES) + (g & 7) * 16
                for d in range(DIM):
                    vals = plsc.load_gather(table_v, [t4 + d])
                    out_v[pl.ds(ob + d * LANES, 16)] = vals
                return carry2

            lax.fori_loop(0, groups, g_body, 0)
            pltpu.sync_copy(out_v, out_hbm.at[pl.ds(base * DIM, chunk * DIM)])
            return carry

        lax.fori_loop(0, n_chunks, chunk_body, 0)

    return emb


def kernel(input_ids, table):
    n_batch, n_seq = input_ids.shape
    n_total = n_batch * n_seq
    ids_sm = jnp.transpose(input_ids, (1, 0)).reshape(-1).astype(jnp.int32)
    table_flat = table.reshape(-1)
    out_flat = _emb_fn(n_total, table.shape[0])(ids_sm, table_flat)
    out4 = out_flat.reshape(n_seq, n_batch // LANES, DIM, LANES)
    return jnp.transpose(out4, (1, 3, 0, 2)).reshape(n_batch, n_seq, DIM)


# speedup vs baseline: 55.3537x; 1.0134x over previous
import functools

import jax
import jax.numpy as jnp
from jax import lax
from jax.experimental import pallas as pl
from jax.experimental.pallas import tpu as pltpu
from jax.experimental.pallas import tpu_sc as plsc

DIM = 4
NUM_WORKERS = 32
LANES = 128
CHUNK = 10240
TILE = DIM * LANES


@functools.lru_cache(maxsize=None)
def _emb_fn(n_total, vocab):
    n_per_w = n_total // NUM_WORKERS
    chunk = CHUNK if n_per_w % CHUNK == 0 else n_per_w
    n_chunks = n_per_w // chunk
    tiles = chunk // LANES
    mesh = plsc.VectorSubcoreMesh(core_axis_name="c", subcore_axis_name="s")

    @functools.partial(
        pl.kernel,
        mesh=mesh,
        out_type=jax.ShapeDtypeStruct((n_total * DIM,), jnp.float32),
        scratch_types=[
            pltpu.VMEM((vocab * DIM,), jnp.float32),
            pltpu.VMEM((2, chunk), jnp.int32),
            pltpu.VMEM((2, chunk * DIM), jnp.float32),
            pltpu.SemaphoreType.DMA((2,)),
            pltpu.SemaphoreType.DMA((2,)),
        ],
        compiler_params=pltpu.CompilerParams(needs_layout_passes=False),
    )
    def emb(ids_hbm, table_hbm, out_hbm, table_v, ids_v, out_v,
            ids_sem, out_sem):
        wid = lax.axis_index("s") * 2 + lax.axis_index("c")
        pltpu.sync_copy(table_hbm, table_v)
        base_w = wid * n_per_w

        def ids_copy(c, b):
            return pltpu.make_async_copy(
                ids_hbm.at[pl.ds(base_w + c * chunk, chunk)],
                ids_v.at[b],
                ids_sem.at[b],
            )

        def out_copy(c, b):
            return pltpu.make_async_copy(
                out_v.at[b],
                out_hbm.at[pl.ds((base_w + c * chunk) * DIM, chunk * DIM)],
                out_sem.at[b],
            )

        def compute(b):
            def t_body(t, carry):
                ib = t * LANES
                ob = t * TILE
                for j in range(LANES // 16):
                    idv = ids_v[b, pl.ds(ib + j * 16, 16)]
                    t4 = idv * DIM
                    for d in range(DIM):
                        vals = plsc.load_gather(table_v, [t4 + d])
                        out_v[b, pl.ds(ob + d * LANES + j * 16, 16)] = vals
                return carry

            lax.fori_loop(0, tiles, t_body, 0)

        ids_copy(0, 0).start()
        if n_chunks > 1:
            ids_copy(1, 1).start()
        for c in range(n_chunks):
            b = c % 2
            if c >= 2:
                out_copy(c - 2, b).wait()
            ids_copy(c, b).wait()
            compute(b)
            out_copy(c, b).start()
            if c + 2 < n_chunks:
                ids_copy(c + 2, b).start()
        for c in range(max(0, n_chunks - 2), n_chunks):
            out_copy(c, c % 2).wait()

    return emb


def kernel(input_ids, table):
    n_batch, n_seq = input_ids.shape
    n_total = n_batch * n_seq
    ids_sm = jnp.transpose(input_ids, (1, 0)).reshape(-1).astype(jnp.int32)
    table_flat = table.reshape(-1)
    out_flat = _emb_fn(n_total, table.shape[0])(ids_sm, table_flat)
    out4 = out_flat.reshape(n_seq, n_batch // LANES, DIM, LANES)
    return jnp.transpose(out4, (1, 3, 0, 2)).reshape(n_batch, n_seq, DIM)


# speedup vs baseline: 134.1047x; 2.4227x over previous
import functools

import jax
import jax.numpy as jnp
from jax import lax
from jax.experimental import pallas as pl
from jax.experimental.pallas import tpu as pltpu
from jax.experimental.pallas import tpu_sc as plsc

DIM = 4
NUM_WORKERS = 32
LANES = 128
CHUNK = 10240
TILE = DIM * LANES


@functools.lru_cache(maxsize=None)
def _emb_fn(n_total, vocab):
    n_per_w = n_total // NUM_WORKERS
    chunk = CHUNK if n_per_w % CHUNK == 0 else n_per_w
    n_chunks = n_per_w // chunk
    tiles = chunk // LANES
    mesh = plsc.VectorSubcoreMesh(core_axis_name="c", subcore_axis_name="s")

    @functools.partial(
        pl.kernel,
        mesh=mesh,
        out_type=jax.ShapeDtypeStruct((n_total * DIM,), jnp.float32),
        scratch_types=[
            pltpu.VMEM((vocab * DIM,), jnp.float32),
            pltpu.VMEM((2, chunk), jnp.int32),
            pltpu.VMEM((2, chunk * DIM), jnp.float32),
            pltpu.SemaphoreType.DMA((2,)),
            pltpu.SemaphoreType.DMA((2,)),
        ],
        compiler_params=pltpu.CompilerParams(needs_layout_passes=False),
    )
    def emb(ids_hbm, table_hbm, out_hbm, table_v, ids_v, out_v,
            ids_sem, out_sem):
        wid = lax.axis_index("s") * 2 + lax.axis_index("c")
        pltpu.sync_copy(table_hbm, table_v)
        base_w = wid * n_per_w

        def ids_copy(c, b):
            return pltpu.make_async_copy(
                ids_hbm.at[pl.ds(base_w + c * chunk, chunk)],
                ids_v.at[b],
                ids_sem.at[b],
            )

        def out_copy(c, b):
            return pltpu.make_async_copy(
                out_v.at[b],
                out_hbm.at[pl.ds((base_w + c * chunk) * DIM, chunk * DIM)],
                out_sem.at[b],
            )

        def compute(b):
            @plsc.parallel_loop(0, chunk // 16, unroll=8)
            def g_body(g):
                idv = ids_v[b, pl.ds(g * 16, 16)]
                t4 = idv * DIM
                ob = (g // 8) * TILE + (g % 8) * 16
                for d in range(DIM):
                    vals = plsc.load_gather(table_v, [t4 + d])
                    out_v[b, pl.ds(ob + d * LANES, 16)] = vals

        ids_copy(0, 0).start()
        if n_chunks > 1:
            ids_copy(1, 1).start()
        for c in range(n_chunks):
            b = c % 2
            if c >= 2:
                out_copy(c - 2, b).wait()
            ids_copy(c, b).wait()
            compute(b)
            out_copy(c, b).start()
            if c + 2 < n_chunks:
                ids_copy(c + 2, b).start()
        for c in range(max(0, n_chunks - 2), n_chunks):
            out_copy(c, c % 2).wait()

    return emb


def kernel(input_ids, table):
    n_batch, n_seq = input_ids.shape
    n_total = n_batch * n_seq
    ids_sm = jnp.transpose(input_ids, (1, 0)).reshape(-1).astype(jnp.int32)
    table_flat = table.reshape(-1)
    out_flat = _emb_fn(n_total, table.shape[0])(ids_sm, table_flat)
    out4 = out_flat.reshape(n_seq, n_batch // LANES, DIM, LANES)
    return jnp.transpose(out4, (1, 3, 0, 2)).reshape(n_batch, n_seq, DIM)


# speedup vs baseline: 163.7216x; 1.2208x over previous
import functools

import jax
import jax.numpy as jnp
from jax import lax
from jax.experimental import pallas as pl
from jax.experimental.pallas import tpu as pltpu
from jax.experimental.pallas import tpu_sc as plsc

DIM = 4
NUM_WORKERS = 32
LANES = 128
CHUNK = 10240
TILE = DIM * LANES


@functools.lru_cache(maxsize=None)
def _emb_fn(n_total, vocab):
    n_per_w = n_total // NUM_WORKERS
    chunk = CHUNK if n_per_w % CHUNK == 0 else n_per_w
    n_chunks = n_per_w // chunk
    tiles = chunk // LANES
    mesh = plsc.VectorSubcoreMesh(core_axis_name="c", subcore_axis_name="s")

    @functools.partial(
        pl.kernel,
        mesh=mesh,
        out_type=jax.ShapeDtypeStruct((n_total * DIM,), jnp.float32),
        scratch_types=[
            pltpu.VMEM((DIM * vocab * 16,), jnp.float32),
            pltpu.VMEM((2, chunk), jnp.int32),
            pltpu.VMEM((2, chunk * DIM), jnp.float32),
            pltpu.SemaphoreType.DMA((2,)),
            pltpu.SemaphoreType.DMA((2,)),
        ],
        compiler_params=pltpu.CompilerParams(needs_layout_passes=False),
    )
    def emb(ids_hbm, table_hbm, out_hbm, table_v, ids_v, out_v,
            ids_sem, out_sem):
        wid = lax.axis_index("s") * 2 + lax.axis_index("c")
        pltpu.sync_copy(table_hbm, table_v)
        base_w = wid * n_per_w
        lane_d = [
            lax.iota(jnp.int32, 16) + d * (vocab * 16) for d in range(DIM)
        ]

        def ids_copy(c, b):
            return pltpu.make_async_copy(
                ids_hbm.at[pl.ds(base_w + c * chunk, chunk)],
                ids_v.at[b],
                ids_sem.at[b],
            )

        def out_copy(c, b):
            return pltpu.make_async_copy(
                out_v.at[b],
                out_hbm.at[pl.ds((base_w + c * chunk) * DIM, chunk * DIM)],
                out_sem.at[b],
            )

        def compute(b):
            @plsc.parallel_loop(0, chunk // 16, unroll=8)
            def g_body(g):
                idv = ids_v[b, pl.ds(g * 16, 16)]
                t16 = idv * 16
                ob = (g // 8) * TILE + (g % 8) * 16
                for d in range(DIM):
                    vals = plsc.load_gather(table_v, [t16 + lane_d[d]])
                    out_v[b, pl.ds(ob + d * LANES, 16)] = vals

        ids_copy(0, 0).start()
        if n_chunks > 1:
            ids_copy(1, 1).start()
        for c in range(n_chunks):
            b = c % 2
            if c >= 2:
                out_copy(c - 2, b).wait()
            ids_copy(c, b).wait()
            compute(b)
            out_copy(c, b).start()
            if c + 2 < n_chunks:
                ids_copy(c + 2, b).start()
        for c in range(max(0, n_chunks - 2), n_chunks):
            out_copy(c, c % 2).wait()

    return emb


def kernel(input_ids, table):
    n_batch, n_seq = input_ids.shape
    n_total = n_batch * n_seq
    ids_sm = jnp.transpose(input_ids, (1, 0)).reshape(-1).astype(jnp.int32)
    table_rep = jnp.broadcast_to(
        table.T[:, :, None], (table.shape[1], table.shape[0], 16)
    ).reshape(-1)
    out_flat = _emb_fn(n_total, table.shape[0])(ids_sm, table_rep)
    out4 = out_flat.reshape(n_seq, n_batch // LANES, DIM, LANES)
    return jnp.transpose(out4, (1, 3, 0, 2)).reshape(n_batch, n_seq, DIM)


# speedup vs baseline: 222.4609x; 1.3588x over previous
import functools

import jax
import jax.numpy as jnp
from jax import lax
from jax.experimental import pallas as pl
from jax.experimental.pallas import tpu as pltpu
from jax.experimental.pallas import tpu_sc as plsc

DIM = 4
NUM_WORKERS = 32
LANES = 128
SUB = 8
TILE = DIM * LANES


@functools.lru_cache(maxsize=None)
def _emb_fn(n_batch, n_seq, vocab):
    n_bt = n_batch // LANES
    n_st = n_seq // SUB
    n_units = n_seq * DIM_Q
    units_per_w = n_units // NUM_WORKERS
    nt_per_u = n_bt // DIM_Q
    chunk_ids = nt_per_u * LANES
    mesh = plsc.VectorSubcoreMesh(core_axis_name="c", subcore_axis_name="s")

    @functools.partial(
        pl.kernel,
        mesh=mesh,
        out_type=jax.ShapeDtypeStruct((n_batch * n_seq * DIM,), jnp.float32),
        scratch_types=[
            pltpu.VMEM((DIM * vocab * 16,), jnp.float32),
            pltpu.VMEM((2, nt_per_u, LANES), jnp.int32),
            pltpu.VMEM((2, chunk_ids * DIM), jnp.float32),
            pltpu.SemaphoreType.DMA((2,)),
            pltpu.SemaphoreType.DMA((2,)),
        ],
        compiler_params=pltpu.CompilerParams(needs_layout_passes=False),
    )
    def emb(ids_hbm, table_hbm, out_hbm, table_v, ids_v, out_v,
            ids_sem, out_sem):
        wid = lax.axis_index("s") * 2 + lax.axis_index("c")
        pltpu.sync_copy(table_hbm, table_v)
        u0 = wid * units_per_w
        lane_d = [
            lax.iota(jnp.int32, 16) + d * (vocab * 16) for d in range(DIM)
        ]

        def ids_copy(u, b):
            s = u // DIM_Q
            q = u % DIM_Q
            return pltpu.make_async_copy(
                ids_hbm.at[s // SUB, pl.ds(q * nt_per_u, nt_per_u), s % SUB, :],
                ids_v.at[b],
                ids_sem.at[b],
            )

        def out_copy(u, b):
            return pltpu.make_async_copy(
                out_v.at[b],
                out_hbm.at[pl.ds(u * chunk_ids * DIM, chunk_ids * DIM)],
                out_sem.at[b],
            )

        def compute(b):
            @plsc.parallel_loop(0, chunk_ids // 16, unroll=8)
            def g_body(g):
                idv = ids_v[b, g // 8, pl.ds((g % 8) * 16, 16)]
                t16 = idv * 16
                ob = (g // 8) * TILE + (g % 8) * 16
                for d in range(DIM):
                    vals = plsc.load_gather(table_v, [t16 + lane_d[d]])
                    out_v[b, pl.ds(ob + d * LANES, 16)] = vals

        def unit_body(k, carry):
            u = u0 + k
            b = k % 2
            out_copy(u - 2, b).wait()
            ids_copy(u, b).wait()
            compute(b)
            out_copy(u, b).start()

            @pl.when(k + 2 < units_per_w)
            def _():
                ids_copy(u + 2, b).start()

            return carry

        ids_copy(u0, 0).start()
        ids_copy(u0 + 1, 1).start()
        for k in (0, 1):
            ids_copy(u0 + k, k).wait()
            compute(k)
            out_copy(u0 + k, k).start()
            ids_copy(u0 + k + 2, k).start()
        lax.fori_loop(2, units_per_w, unit_body, 0)
        for k in (units_per_w - 2, units_per_w - 1):
            out_copy(u0 + k, k % 2).wait()

    return emb


DIM_Q = 4


def kernel(input_ids, table):
    n_batch, n_seq = input_ids.shape
    ids4 = input_ids.astype(jnp.int32).reshape(
        n_batch // LANES, LANES, n_seq // SUB, SUB
    )
    ids_raw = jnp.transpose(ids4, (2, 0, 3, 1))
    table_rep = jnp.broadcast_to(
        table.T[:, :, None], (table.shape[1], table.shape[0], 16)
    ).reshape(-1)
    out_flat = _emb_fn(n_batch, n_seq, table.shape[0])(ids_raw, table_rep)
    out4 = out_flat.reshape(n_seq, n_batch // LANES, DIM, LANES)
    return jnp.transpose(out4, (1, 3, 0, 2)).reshape(n_batch, n_seq, DIM)
